# baseline (speedup 1.0000x reference)
import jax
import jax.numpy as jnp
from jax.experimental import pallas as pl
from jax.experimental.pallas import tpu as pltpu

_NSPLIT = 8


def _cdfg_kernel(idx_ref, xs_ref, a0_ref, a1_ref, a2_ref, a3_ref,
                 a4_ref, a5_ref, a6_ref, a7_ref, m_ref,
                 win_ref, bin_ref, w1_ref, b1_ref, w2_ref, b2_ref, out_ref):
    b = pl.program_id(0)
    xs = xs_ref[0]
    m = m_ref[b][None, :]
    a_parts = (a0_ref, a1_ref, a2_ref, a3_ref, a4_ref, a5_ref, a6_ref, a7_ref)

    def conv(y):
        return jnp.concatenate(
            [jnp.dot(p[0], y, preferred_element_type=jnp.float32)
             for p in a_parts], axis=0)

    x0 = jnp.maximum(
        jnp.dot(xs, win_ref[...], preferred_element_type=jnp.float32)
        + bin_ref[...], 0.0)
    y1 = jnp.dot(x0, w1_ref[...], preferred_element_type=jnp.float32)
    x1 = jnp.maximum(conv(y1) + b1_ref[...], 0.0)
    y2 = jnp.dot(x1, w2_ref[...], preferred_element_type=jnp.float32)
    x2 = jnp.tanh(conv(y2) + b2_ref[...])
    x = x2 + x0
    num = jnp.dot(m, x, preferred_element_type=jnp.float32)
    den = jnp.sum(m)
    out_ref[b, :] = (num / den)[0]


def kernel(graph, coverpoint, coverpoint_mask, batch_xs, batch_as,
           W_in, b_in, W1, b1, W2, b2):
    B = graph.shape[0]
    _, N, F = batch_xs.shape
    H = W1.shape[1]
    NQ = N // _NSPLIT

    a_specs = [
        pl.BlockSpec((1, NQ, N), lambda b, i, q=q: (i[b, 0], q, 0))
        for q in range(_NSPLIT)
    ]
    grid_spec = pltpu.PrefetchScalarGridSpec(
        num_scalar_prefetch=1,
        grid=(B,),
        in_specs=[
            pl.BlockSpec((1, N, F), lambda b, i: (i[b, 0], 0, 0)),
            *a_specs,
            pl.BlockSpec((B, N), lambda b, i: (0, 0)),
            pl.BlockSpec((F, H), lambda b, i: (0, 0)),
            pl.BlockSpec((1, H), lambda b, i: (0, 0)),
            pl.BlockSpec((H, H), lambda b, i: (0, 0)),
            pl.BlockSpec((1, H), lambda b, i: (0, 0)),
            pl.BlockSpec((H, H), lambda b, i: (0, 0)),
            pl.BlockSpec((1, H), lambda b, i: (0, 0)),
        ],
        out_specs=pl.BlockSpec((B, H), lambda b, i: (0, 0)),
    )
    return pl.pallas_call(
        _cdfg_kernel,
        grid_spec=grid_spec,
        out_shape=jax.ShapeDtypeStruct((B, H), jnp.float32),
        compiler_params=pltpu.CompilerParams(
            vmem_limit_bytes=100 * 1024 * 1024),
    )(graph, batch_xs, *([batch_as] * _NSPLIT),
      coverpoint_mask.astype(jnp.float32),
      W_in, b_in.reshape(1, -1), W1, b1.reshape(1, -1), W2, b2.reshape(1, -1))


# speedup vs baseline: 1.0345x; 1.0345x over previous
import jax
import jax.numpy as jnp
from jax.experimental import pallas as pl
from jax.experimental.pallas import tpu as pltpu

_NSPLIT = 2


def _cdfg_kernel(idx_ref, xs_ref, a0_ref, a1_ref, m_ref,
                 win_ref, bin_ref, w1_ref, b1_ref, w2_ref, b2_ref, out_ref):
    b = pl.program_id(0)
    xs = xs_ref[0]
    m = m_ref[b][None, :]
    a_parts = (a0_ref, a1_ref)

    def conv(y):
        return jnp.concatenate(
            [jnp.dot(p[0], y, preferred_element_type=jnp.float32)
             for p in a_parts], axis=0)

    x0 = jnp.maximum(
        jnp.dot(xs, win_ref[...], preferred_element_type=jnp.float32)
        + bin_ref[...], 0.0)
    y1 = jnp.dot(x0, w1_ref[...], preferred_element_type=jnp.float32)
    x1 = jnp.maximum(conv(y1) + b1_ref[...], 0.0)
    y2 = jnp.dot(x1, w2_ref[...], preferred_element_type=jnp.float32)
    x2 = jnp.tanh(conv(y2) + b2_ref[...])
    x = x2 + x0
    num = jnp.dot(m, x, preferred_element_type=jnp.float32)
    den = jnp.sum(m)
    out_ref[b, :] = (num / den)[0]


def kernel(graph, coverpoint, coverpoint_mask, batch_xs, batch_as,
           W_in, b_in, W1, b1, W2, b2):
    B = graph.shape[0]
    _, N, F = batch_xs.shape
    H = W1.shape[1]
    NQ = N // _NSPLIT

    a_specs = [
        pl.BlockSpec((1, NQ, N), lambda b, i, q=q: (i[b, 0], q, 0))
        for q in range(_NSPLIT)
    ]
    grid_spec = pltpu.PrefetchScalarGridSpec(
        num_scalar_prefetch=1,
        grid=(B,),
        in_specs=[
            pl.BlockSpec((1, N, F), lambda b, i: (i[b, 0], 0, 0)),
            *a_specs,
            pl.BlockSpec((B, N), lambda b, i: (0, 0)),
            pl.BlockSpec((F, H), lambda b, i: (0, 0)),
            pl.BlockSpec((1, H), lambda b, i: (0, 0)),
            pl.BlockSpec((H, H), lambda b, i: (0, 0)),
            pl.BlockSpec((1, H), lambda b, i: (0, 0)),
            pl.BlockSpec((H, H), lambda b, i: (0, 0)),
            pl.BlockSpec((1, H), lambda b, i: (0, 0)),
        ],
        out_specs=pl.BlockSpec((B, H), lambda b, i: (0, 0)),
    )
    return pl.pallas_call(
        _cdfg_kernel,
        grid_spec=grid_spec,
        out_shape=jax.ShapeDtypeStruct((B, H), jnp.float32),
        compiler_params=pltpu.CompilerParams(
            vmem_limit_bytes=100 * 1024 * 1024),
    )(graph, batch_xs, *([batch_as] * _NSPLIT),
      coverpoint_mask.astype(jnp.float32),
      W_in, b_in.reshape(1, -1), W1, b1.reshape(1, -1), W2, b2.reshape(1, -1))
